# baseline (speedup 1.0000x reference)
import functools

import jax
import jax.numpy as jnp
from jax.experimental import pallas as pl
from jax.experimental.pallas import tpu as pltpu

NUM_TOKENS = 32768
HIDDEN = 768
NUM_EXPERTS = 64

BLOCK = 1024


def _router_block(x_ref, wt_ref, scores_ref, w_ref, i_ref):
    logits = jnp.dot(x_ref[...], wt_ref[...], preferred_element_type=jnp.float32)
    m = jnp.max(logits, axis=-1, keepdims=True)
    e = jnp.exp(logits - m)
    s = jnp.sum(e, axis=-1, keepdims=True)
    scores_ref[...] = e / s
    w_ref[...] = 1.0 / s[:, 0]
    i_ref[...] = jnp.argmax(logits, axis=-1).astype(jnp.int32)


@jax.jit
def _router(x, Wt):
    n_blocks = NUM_TOKENS // BLOCK
    scores, w, idx = pl.pallas_call(
        _router_block,
        grid=(n_blocks,),
        in_specs=[
            pl.BlockSpec((BLOCK, HIDDEN), lambda i: (i, 0)),
            pl.BlockSpec((HIDDEN, NUM_EXPERTS), lambda i: (0, 0)),
        ],
        out_specs=[
            pl.BlockSpec((BLOCK, NUM_EXPERTS), lambda i: (i, 0)),
            pl.BlockSpec((BLOCK,), lambda i: (i,)),
            pl.BlockSpec((BLOCK,), lambda i: (i,)),
        ],
        out_shape=[
            jax.ShapeDtypeStruct((NUM_TOKENS, NUM_EXPERTS), jnp.float32),
            jax.ShapeDtypeStruct((NUM_TOKENS,), jnp.float32),
            jax.ShapeDtypeStruct((NUM_TOKENS,), jnp.int32),
        ],
        compiler_params=pltpu.CompilerParams(
            dimension_semantics=("parallel",),
        ),
    )(x, Wt)
    return w[:, None], idx[:, None], scores


def kernel(x, W):
    x2 = x.reshape(-1, x.shape[-1])
    w, idx, scores = _router(x2, W.T)
    return (w, idx, scores)


# speedup vs baseline: 1.1679x; 1.1679x over previous
import functools

import jax
import jax.numpy as jnp
from jax.experimental import pallas as pl
from jax.experimental.pallas import tpu as pltpu

NUM_TOKENS = 32768
HIDDEN = 768
NUM_EXPERTS = 64

BLOCK = 1024


def _router_block(x_ref, wt_ref, scores_ref, w_ref, i_ref):
    logits = jnp.dot(x_ref[...], wt_ref[...], preferred_element_type=jnp.float32)
    m = jnp.max(logits, axis=-1, keepdims=True)
    e = jnp.exp(logits - m)
    s = jnp.sum(e, axis=-1, keepdims=True)
    scores_ref[...] = e / s
    w_ref[...] = 1.0 / s
    lane = jax.lax.broadcasted_iota(jnp.int32, logits.shape, 1).astype(jnp.float32)
    hit = jnp.where(logits == m, lane, float(NUM_EXPERTS))
    i_ref[...] = jnp.min(hit, axis=-1, keepdims=True).astype(jnp.int32)


@jax.jit
def _router(x, Wt):
    n_blocks = NUM_TOKENS // BLOCK
    scores, w, idx = pl.pallas_call(
        _router_block,
        grid=(n_blocks,),
        in_specs=[
            pl.BlockSpec((BLOCK, HIDDEN), lambda i: (i, 0)),
            pl.BlockSpec((HIDDEN, NUM_EXPERTS), lambda i: (0, 0)),
        ],
        out_specs=[
            pl.BlockSpec((BLOCK, NUM_EXPERTS), lambda i: (i, 0)),
            pl.BlockSpec((BLOCK, 1), lambda i: (i, 0)),
            pl.BlockSpec((BLOCK, 1), lambda i: (i, 0)),
        ],
        out_shape=[
            jax.ShapeDtypeStruct((NUM_TOKENS, NUM_EXPERTS), jnp.float32),
            jax.ShapeDtypeStruct((NUM_TOKENS, 1), jnp.float32),
            jax.ShapeDtypeStruct((NUM_TOKENS, 1), jnp.int32),
        ],
        compiler_params=pltpu.CompilerParams(
            dimension_semantics=("parallel",),
        ),
    )(x, Wt)
    return w, idx, scores


def kernel(x, W):
    x2 = x.reshape(-1, x.shape[-1])
    w, idx, scores = _router(x2, W.T)
    return (w, idx, scores)
